# baseline (speedup 1.0000x reference)
import functools

import jax
import jax.numpy as jnp
from jax import lax
from jax.experimental import pallas as pl
from jax.experimental.pallas import tpu as pltpu
from jax.experimental.pallas import tpu_sc as plsc

L = 16


def _splat(ref, i):
    return plsc.load_gather(ref, [jnp.full((L,), i, jnp.int32)])


def kernel(uv, intrinsics, extrinsics, size):
    S, V, _, P = uv.shape
    info = plsc.get_sparse_core_info()
    NC, NS = info.num_cores, info.num_subcores
    NW = NC * NS
    SPAN = P // NW
    CH = 10000
    NCHUNK = SPAN // CH
    NV = S * V

    mesh = plsc.VectorSubcoreMesh(core_axis_name="c", subcore_axis_name="s")

    @functools.partial(
        pl.kernel,
        out_type=(
            jax.ShapeDtypeStruct((S * V * 3 * P,), jnp.float32),
            jax.ShapeDtypeStruct((S * V * 3,), jnp.float32),
            jax.ShapeDtypeStruct((S * V * 2 * P,), jnp.float32),
        ),
        mesh=mesh,
        scratch_types=[
            pltpu.VMEM((CH,), jnp.float32),
            pltpu.VMEM((CH,), jnp.float32),
            pltpu.VMEM((3 * CH,), jnp.float32),
            pltpu.VMEM((S * L,), jnp.float32),
            pltpu.VMEM((S * V * L,), jnp.float32),
            pltpu.VMEM((S * V * 3,), jnp.float32),
        ],
        compiler_params=pltpu.CompilerParams(needs_layout_passes=False),
    )
    def run(uv_h, intr_h, ext_h, out_h, rs_h, uvc_h, u_v, w_v, out_v, intr_v, ext_v, rs_v):
        wid = lax.axis_index("s") * NC + lax.axis_index("c")
        pltpu.sync_copy(intr_h, intr_v)
        pltpu.sync_copy(ext_h, ext_v)

        @pl.when(wid == 0)
        def _():
            lane = lax.iota(jnp.int32, L)
            for half in range(2):
                pos = lane + half * L
                src = jnp.minimum((pos // 3) * L + (pos % 3) * 4 + 3, S * V * L - 1)
                vals = plsc.load_gather(ext_v, [src])
                dst = jnp.minimum(pos, S * V * 3 - 1)
                msk = pos < S * V * 3
                plsc.store_scatter(rs_v, [dst], vals, mask=msk)
            pltpu.sync_copy(rs_v, rs_h)

        base_p = wid * SPAN
        lane3 = lax.iota(jnp.int32, L) * 3

        def sv_loop(sv, carry):
            s = sv // V
            ib = s * L
            eb = sv * L
            rfx = 1.0 / _splat(intr_v, ib + 0)
            rfy = 1.0 / _splat(intr_v, ib + 5)
            cx = _splat(intr_v, ib + 2)
            cy = _splat(intr_v, ib + 6)

            C = []
            for k in range(3):
                c0 = _splat(ext_v, eb + 4 * k + 0) * rfx
                c1 = _splat(ext_v, eb + 4 * k + 1) * rfy
                c2 = _splat(ext_v, eb + 4 * k + 2) - c0 * cx - c1 * cy
                C.append((c0, c1, c2))

            uv_base = sv * 2 * P + base_p
            out_base = sv * 3 * P + 3 * base_p

            def ch_loop(c, carry2):
                pltpu.sync_copy(uv_h.at[pl.ds(uv_base + c * CH, CH)], u_v)
                pltpu.sync_copy(uv_h.at[pl.ds(uv_base + P + c * CH, CH)], w_v)

                def inner(i, carry3):
                    off = i * L
                    u = u_v[pl.ds(off, L)]
                    w = w_v[pl.ds(off, L)]
                    d0 = C[0][2] + u * C[0][0] + w * C[0][1]
                    d1 = C[1][2] + u * C[1][0] + w * C[1][1]
                    d2 = C[2][2] + u * C[2][0] + w * C[2][1]
                    ss = d0 * d0 + d1 * d1 + d2 * d2
                    yb = 0x5F3759DF - lax.shift_right_logical(
                        lax.bitcast_convert_type(ss, jnp.int32), 1
                    )
                    y = lax.bitcast_convert_type(yb, jnp.float32)
                    nh = ss * -0.5
                    y = y * (1.5 + nh * y * y)
                    y = y * (1.5 + nh * y * y)
                    y = y * (1.5 + nh * y * y)
                    y = jnp.minimum(y, 1e12)
                    idx = lane3 + i * (3 * L)
                    plsc.store_scatter(out_v, [idx], d0 * y)
                    plsc.store_scatter(out_v, [idx + 1], d1 * y)
                    plsc.store_scatter(out_v, [idx + 2], d2 * y)
                    return carry3

                lax.fori_loop(0, CH // L, inner, 0)
                pltpu.sync_copy(out_v, out_h.at[pl.ds(out_base + c * 3 * CH, 3 * CH)])
                pltpu.sync_copy(u_v, uvc_h.at[pl.ds(uv_base + c * CH, CH)])
                pltpu.sync_copy(w_v, uvc_h.at[pl.ds(uv_base + P + c * CH, CH)])
                return carry2

            lax.fori_loop(0, NCHUNK, ch_loop, 0)
            return carry

        lax.fori_loop(0, NV, sv_loop, 0)

    ray_flat, rs_buf, uv_copy = run(
        uv.reshape(-1),
        intrinsics.reshape(-1),
        extrinsics.reshape(-1),
    )
    ray_dir = ray_flat.reshape(S, V, P, 3)
    ray_start = rs_buf.reshape(S, V, 1, 3)
    uv_out = uv_copy.reshape(S, V, 2, P, 1, 1)
    return (ray_start, ray_dir, uv_out)


# speedup vs baseline: 8.7113x; 8.7113x over previous
import functools

import jax
import jax.numpy as jnp
from jax import lax
from jax.experimental import pallas as pl
from jax.experimental.pallas import tpu as pltpu
from jax.experimental.pallas import tpu_sc as plsc

L = 16


def _splat(ref, i):
    return plsc.load_gather(ref, [jnp.full((L,), i, jnp.int32)])


def kernel(uv, intrinsics, extrinsics, size):
    S, V, _, P = uv.shape
    info = plsc.get_sparse_core_info()
    NC, NS = info.num_cores, info.num_subcores
    NW = NC * NS
    PC = P // 128
    NPC = 25
    CHP = NPC * 128
    CPS = PC // NPC
    NCH = S * CPS
    NV = S * V

    mesh = plsc.VectorSubcoreMesh(core_axis_name="c", subcore_axis_name="s")

    @functools.partial(
        pl.kernel,
        out_type=(
            jax.ShapeDtypeStruct((S * 3 * V * P,), jnp.float32),
            jax.ShapeDtypeStruct((S * V * 3,), jnp.float32),
            jax.ShapeDtypeStruct((S * V * 2 * P,), jnp.float32),
        ),
        mesh=mesh,
        scratch_types=[
            [pltpu.VMEM((1, 1, 2, CHP), jnp.float32) for _ in range(4)],
            [pltpu.VMEM((NPC * 4 * 128,), jnp.float32) for _ in range(3)],
            pltpu.VMEM((CHP,), jnp.float32),
            pltpu.VMEM((CHP,), jnp.float32),
            pltpu.VMEM((S * L,), jnp.float32),
            pltpu.VMEM((NV * L,), jnp.float32),
            pltpu.VMEM((S * V * 3,), jnp.float32),
        ],
        compiler_params=pltpu.CompilerParams(needs_layout_passes=False),
    )
    def run(uv_h, intr_h, ext_h, ray_h, rs_h, uvc_h,
            in_bufs, ray_bufs, uvu_v, uvw_v, intr_v, ext_v, rs_v):
        wid = lax.axis_index("s") * NC + lax.axis_index("c")
        pltpu.sync_copy(intr_h, intr_v)
        pltpu.sync_copy(ext_h, ext_v)

        @pl.when(wid == 0)
        def _():
            lane = lax.iota(jnp.int32, L)
            for half in range(2):
                pos = lane + half * L
                sj = pos // 12
                kj = (pos % 12) // 4
                vj = pos % 4
                src = jnp.minimum((sj * V + vj) * L + 4 * kj + 3, NV * L - 1)
                vals = plsc.load_gather(ext_v, [src])
                dst = jnp.minimum(pos, S * V * 3 - 1)
                msk = pos < S * V * 3
                plsc.store_scatter(rs_v, [dst], vals, mask=msk)
            pltpu.sync_copy(rs_v, rs_h)

        njobs = (NCH - 1 - wid) // NW + 1

        def chunk_body(j, carry):
            t = wid + j * NW
            s = t // CPS
            pc0 = (t % CPS) * NPC
            p0 = pc0 * 128

            for v in range(V):
                pltpu.sync_copy(
                    uv_h.at[pl.ds(s, 1), pl.ds(v, 1), :, pl.ds(p0, CHP)],
                    in_bufs[v],
                )

            ib = s * L
            rfx = 1.0 / _splat(intr_v, ib + 0)
            rfy = 1.0 / _splat(intr_v, ib + 5)
            cx = _splat(intr_v, ib + 2)
            cy = _splat(intr_v, ib + 6)

            for v in range(V):
                eb = (s * V + v) * L
                C = []
                for k in range(3):
                    c0 = _splat(ext_v, eb + 4 * k + 0) * rfx
                    c1 = _splat(ext_v, eb + 4 * k + 1) * rfy
                    c2 = _splat(ext_v, eb + 4 * k + 2) - c0 * cx - c1 * cy
                    C.append((c0, c1, c2))
                inb = in_bufs[v]

                def inner(i, carry2, v=v, C=C, inb=inb):
                    off = i * L
                    u = inb[0, 0, 0, pl.ds(off, L)]
                    w = inb[0, 0, 1, pl.ds(off, L)]
                    uvu_v[pl.ds(off, L)] = u
                    uvw_v[pl.ds(off, L)] = w
                    d0 = C[0][2] + u * C[0][0] + w * C[0][1]
                    d1 = C[1][2] + u * C[1][0] + w * C[1][1]
                    d2 = C[2][2] + u * C[2][0] + w * C[2][1]
                    ss = d0 * d0 + d1 * d1 + d2 * d2
                    yb = 0x5F3759DF - lax.shift_right_logical(
                        lax.bitcast_convert_type(ss, jnp.int32), 1
                    )
                    y = lax.bitcast_convert_type(yb, jnp.float32)
                    nh = ss * -0.5
                    y = y * (1.5 + nh * y * y)
                    y = y * (1.5 + nh * y * y)
                    y = y * (1.5 + nh * y * y)
                    y = jnp.minimum(y, 1e12)
                    dst = (i // 8) * (V * 128) + v * 128 + (i % 8) * L
                    ray_bufs[0][pl.ds(dst, L)] = d0 * y
                    ray_bufs[1][pl.ds(dst, L)] = d1 * y
                    ray_bufs[2][pl.ds(dst, L)] = d2 * y
                    return carry2

                lax.fori_loop(0, CHP // L, inner, 0)

                base_u = ((s * V + v) * 2 + 0) * P + p0
                base_w = ((s * V + v) * 2 + 1) * P + p0
                pltpu.sync_copy(uvu_v, uvc_h.at[pl.ds(base_u, CHP)])
                pltpu.sync_copy(uvw_v, uvc_h.at[pl.ds(base_w, CHP)])

            for k in range(3):
                base = ((s * 3 + k) * PC + pc0) * (V * 128)
                pltpu.sync_copy(ray_bufs[k], ray_h.at[pl.ds(base, NPC * V * 128)])
            return carry

        lax.fori_loop(0, njobs, chunk_body, 0)

    ray_flat, rs_buf, uv_copy = run(uv, intrinsics.reshape(-1), extrinsics.reshape(-1))
    ray_dir = (
        ray_flat.reshape(S, 3, PC, V, 128)
        .transpose(0, 3, 2, 4, 1)
        .reshape(S, V, P, 3)
    )
    ray_start = rs_buf.reshape(S, 3, 1, V).transpose(0, 3, 2, 1)
    uv_out = uv_copy.reshape(S, V, 2, P, 1, 1)
    return (ray_start, ray_dir, uv_out)
